# baseline (speedup 1.0000x reference)
import functools
import math

import jax
import jax.numpy as jnp
from jax import lax
from jax.experimental import pallas as pl
from jax.experimental.pallas import tpu as pltpu
from jax.experimental.pallas import tpu_sc as plsc

BATCH = 16384
EMB = 64

_NC = 2
_NS = 16
_NW = _NC * _NS


_IDS_CHUNK = 2048


def _gather_body(uids_hbm, iids_hbm, ut_hbm, it_hbm, ue_hbm, ie_hbm,
                 uidx_s, iidx_s, sem, half):
    cid = lax.axis_index("c")
    base = cid * half

    def do_chunk(c, carry):
        off = base + c * _IDS_CHUNK
        pltpu.sync_copy(uids_hbm.at[pl.ds(off, _IDS_CHUNK)], uidx_s)
        pltpu.sync_copy(iids_hbm.at[pl.ds(off, _IDS_CHUNK)], iidx_s)

        def issue(i, carry2):
            pltpu.async_copy(ut_hbm.at[pl.ds(uidx_s[i], 1)],
                             ue_hbm.at[pl.ds(off + i, 1)], sem)
            pltpu.async_copy(it_hbm.at[pl.ds(iidx_s[i], 1)],
                             ie_hbm.at[pl.ds(off + i, 1)], sem)
            return carry2

        lax.fori_loop(0, _IDS_CHUNK, issue, 0)
        return carry

    lax.fori_loop(0, half // _IDS_CHUNK, do_chunk, 0)
    pltpu.make_async_copy(ut_hbm.at[pl.ds(0, BATCH)], ue_hbm, sem).wait()


def _sc_gather(user_ids, item_ids, user_table, item_table):
    half = BATCH // _NC
    mesh = plsc.ScalarSubcoreMesh(axis_name="c", num_cores=_NC)
    out_type = [
        jax.ShapeDtypeStruct((BATCH, EMB), jnp.float32),
        jax.ShapeDtypeStruct((BATCH, EMB), jnp.float32),
    ]
    scratch = [
        pltpu.SMEM((_IDS_CHUNK,), jnp.int32),
        pltpu.SMEM((_IDS_CHUNK,), jnp.int32),
        pltpu.SemaphoreType.DMA,
    ]
    k = pl.kernel(
        functools.partial(_gather_body, half=half),
        out_type=out_type,
        mesh=mesh,
        scratch_types=scratch,
    )
    return k(user_ids, item_ids, user_table, item_table)


def _layernorm(x, g, b, eps=1e-5):
    mu = jnp.mean(x, axis=-1, keepdims=True)
    var = jnp.mean((x - mu) ** 2, axis=-1, keepdims=True)
    return (x - mu) / jnp.sqrt(var + eps) * g + b


def _gelu(x):
    return 0.5 * x * (1.0 + lax.erf(x * (1.0 / math.sqrt(2.0))))


def _mlp_body(ue, ie, W0u, W0i, b0, g0, beta0, W1, b1, g1, beta1,
              W2, b2, g2, beta2, W_out, b_out, out):
    dot = functools.partial(jnp.dot, preferred_element_type=jnp.float32,
                            precision=lax.Precision.HIGHEST)
    x = dot(ue[...], W0u[...]) + dot(ie[...], W0i[...]) + b0[...]
    x = _gelu(_layernorm(x, g0[...], beta0[...]))
    x = dot(x, W1[...]) + b1[...]
    x = _gelu(_layernorm(x, g1[...], beta1[...]))
    x = dot(x, W2[...]) + b2[...]
    x = _gelu(_layernorm(x, g2[...], beta2[...]))
    out[...] = dot(x, W_out[...]) + b_out[...]


def _tc_mlp(ue, ie, W0, b0, g0, beta0, W1, b1, g1, beta1,
            W2, b2, g2, beta2, W_out, b_out):
    blk = 2048
    grid = (BATCH // blk,)
    W0u = W0[:EMB]
    W0i = W0[EMB:]

    def row_spec(n):
        return pl.BlockSpec((blk, n), lambda i: (i, 0))

    def full_spec(a):
        return pl.BlockSpec(a.shape, lambda i: (0,) * a.ndim)

    b0r, g0r, beta0r = (a.reshape(1, -1) for a in (b0, g0, beta0))
    b1r, g1r, beta1r = (a.reshape(1, -1) for a in (b1, g1, beta1))
    b2r, g2r, beta2r = (a.reshape(1, -1) for a in (b2, g2, beta2))
    b_outr = b_out.reshape(1, -1)

    args = (ue, ie, W0u, W0i, b0r, g0r, beta0r, W1, b1r, g1r, beta1r,
            W2, b2r, g2r, beta2r, W_out, b_outr)
    in_specs = [row_spec(EMB), row_spec(EMB)] + [full_spec(a) for a in args[2:]]
    return pl.pallas_call(
        _mlp_body,
        grid=grid,
        in_specs=in_specs,
        out_specs=pl.BlockSpec((blk, 1), lambda i: (i, 0)),
        out_shape=jax.ShapeDtypeStruct((BATCH, 1), jnp.float32),
    )(*args)


def kernel(user_ids, item_ids, user_table, item_table,
           W0, b0, g0, beta0, W1, b1, g1, beta1, W2, b2, g2, beta2,
           W_out, b_out):
    ue, ie = _sc_gather(user_ids.astype(jnp.int32), item_ids.astype(jnp.int32),
                        user_table, item_table)
    return _tc_mlp(ue, ie, W0, b0, g0, beta0, W1, b1, g1, beta1,
                   W2, b2, g2, beta2, W_out, b_out)


# speedup vs baseline: 1.0984x; 1.0984x over previous
import functools
import math

import jax
import jax.numpy as jnp
from jax import lax
from jax.experimental import pallas as pl
from jax.experimental.pallas import tpu as pltpu
from jax.experimental.pallas import tpu_sc as plsc

BATCH = 16384
EMB = 64

_NC = 2
_NS = 16
_NW = _NC * _NS


def _gather_body(uids_hbm, iids_hbm, ut_hbm, it_hbm, x_hbm,
                 uidx_v, iidx_v, u2_v, i2_v, urows_v, irows_v,
                 sem_u, sem_i, bpw):
    wid = lax.axis_index("s") * _NC + lax.axis_index("c")
    base = wid * bpw
    pltpu.sync_copy(uids_hbm.at[pl.ds(base, bpw)], uidx_v)
    pltpu.sync_copy(iids_hbm.at[pl.ds(base, bpw)], iidx_v)
    for j in range(bpw // 16):
        s = pl.ds(16 * j, 16)
        u2_v[s] = uidx_v[s]
        i2_v[s] = iidx_v[s]
    cu = pltpu.async_copy(ut_hbm.at[u2_v], urows_v, sem_u)
    ci = pltpu.async_copy(it_hbm.at[i2_v], irows_v, sem_i)
    cu.wait()
    pltpu.sync_copy(urows_v, x_hbm.at[pl.ds(base, bpw), pl.ds(0, EMB)])
    ci.wait()
    pltpu.sync_copy(irows_v, x_hbm.at[pl.ds(base, bpw), pl.ds(EMB, EMB)])


def _sc_gather(user_ids, item_ids, user_table, item_table):
    bpw = BATCH // _NW
    mesh = plsc.VectorSubcoreMesh(core_axis_name="c", subcore_axis_name="s")
    out_type = jax.ShapeDtypeStruct((BATCH, 2 * EMB), jnp.float32)
    scratch = [
        pltpu.VMEM((bpw,), jnp.int32),
        pltpu.VMEM((bpw,), jnp.int32),
        pltpu.VMEM((bpw,), jnp.int32),
        pltpu.VMEM((bpw,), jnp.int32),
        pltpu.VMEM((bpw, EMB), jnp.float32),
        pltpu.VMEM((bpw, EMB), jnp.float32),
        pltpu.SemaphoreType.DMA,
        pltpu.SemaphoreType.DMA,
    ]
    k = pl.kernel(
        functools.partial(_gather_body, bpw=bpw),
        out_type=out_type,
        mesh=mesh,
        scratch_types=scratch,
        compiler_params=pltpu.CompilerParams(
            use_tc_tiling_on_sc=False,
            needs_layout_passes=False,
        ),
    )
    return k(user_ids, item_ids, user_table, item_table)


def _layernorm(x, g, b, eps=1e-5):
    mu = jnp.mean(x, axis=-1, keepdims=True)
    var = jnp.mean((x - mu) ** 2, axis=-1, keepdims=True)
    return (x - mu) / jnp.sqrt(var + eps) * g + b


def _gelu(x):
    return 0.5 * x * (1.0 + lax.erf(x * (1.0 / math.sqrt(2.0))))


def _mlp_body(xin, W0, b0, g0, beta0, W1, b1, g1, beta1,
              W2, b2, g2, beta2, W_out, b_out, out):
    dot = functools.partial(jnp.dot, preferred_element_type=jnp.float32,
                            precision=lax.Precision.HIGHEST)
    x = dot(xin[...], W0[...]) + b0[...]
    x = _gelu(_layernorm(x, g0[...], beta0[...]))
    x = dot(x, W1[...]) + b1[...]
    x = _gelu(_layernorm(x, g1[...], beta1[...]))
    x = dot(x, W2[...]) + b2[...]
    x = _gelu(_layernorm(x, g2[...], beta2[...]))
    out[...] = dot(x, W_out[...]) + b_out[...]


def _tc_mlp(x, W0, b0, g0, beta0, W1, b1, g1, beta1,
            W2, b2, g2, beta2, W_out, b_out):
    blk = 2048
    grid = (BATCH // blk,)

    def row_spec(n):
        return pl.BlockSpec((blk, n), lambda i: (i, 0))

    def full_spec(a):
        return pl.BlockSpec(a.shape, lambda i: (0,) * a.ndim)

    b0r, g0r, beta0r = (a.reshape(1, -1) for a in (b0, g0, beta0))
    b1r, g1r, beta1r = (a.reshape(1, -1) for a in (b1, g1, beta1))
    b2r, g2r, beta2r = (a.reshape(1, -1) for a in (b2, g2, beta2))
    b_outr = b_out.reshape(1, -1)

    args = (x, W0, b0r, g0r, beta0r, W1, b1r, g1r, beta1r,
            W2, b2r, g2r, beta2r, W_out, b_outr)
    in_specs = [row_spec(2 * EMB)] + [full_spec(a) for a in args[1:]]
    return pl.pallas_call(
        _mlp_body,
        grid=grid,
        in_specs=in_specs,
        out_specs=pl.BlockSpec((blk, 1), lambda i: (i, 0)),
        out_shape=jax.ShapeDtypeStruct((BATCH, 1), jnp.float32),
    )(*args)


def kernel(user_ids, item_ids, user_table, item_table,
           W0, b0, g0, beta0, W1, b1, g1, beta1, W2, b2, g2, beta2,
           W_out, b_out):
    x = _sc_gather(user_ids.astype(jnp.int32), item_ids.astype(jnp.int32),
                   user_table, item_table)
    return _tc_mlp(x, W0, b0, g0, beta0, W1, b1, g1, beta1,
                   W2, b2, g2, beta2, W_out, b_out)


# speedup vs baseline: 1.7106x; 1.5573x over previous
import functools

import jax
import jax.numpy as jnp
from jax import lax
from jax.experimental import pallas as pl
from jax.experimental.pallas import tpu as pltpu
from jax.experimental.pallas import tpu_sc as plsc

BATCH = 16384
EMB = 64

_NC = 2
_NS = 16
_NW = _NC * _NS

_TBLK = 2048


def _detrans_body(t1_ref, t2_ref, o1_ref, o2_ref):
    for t_ref, o_ref in ((t1_ref, o1_ref), (t2_ref, o2_ref)):
        y = t_ref[...].T
        h = _TBLK // 2
        o_ref[...] = jnp.concatenate([y[:h], y[h:]], axis=1)


def _detranspose(t1T, t2T):
    n = t1T.shape[1]
    grid = (pl.cdiv(n, _TBLK),)
    in_spec = pl.BlockSpec((EMB, _TBLK), lambda i: (0, i))
    out_spec = pl.BlockSpec((_TBLK // 2, 2 * EMB), lambda i: (i, 0))
    nout = (_TBLK // 2) * pl.cdiv(n, _TBLK)
    out_shape = jax.ShapeDtypeStruct((nout, 2 * EMB), jnp.float32)
    return pl.pallas_call(
        _detrans_body,
        grid=grid,
        in_specs=[in_spec, in_spec],
        out_specs=[out_spec, out_spec],
        out_shape=[out_shape, out_shape],
    )(t1T, t2T)


_CH = 256


def _gather_body(uids_hbm, iids_hbm, t1_hbm, t2_hbm, x_hbm,
                 uidx_v, iidx_v, uhalf_v, ihalf_v, upar_v, ipar_v,
                 urows_v, irows_v, cat_v, sem_u, sem_i, bpw):
    wid = lax.axis_index("s") * _NC + lax.axis_index("c")
    base = wid * bpw
    pltpu.sync_copy(uids_hbm.at[pl.ds(base, bpw)], uidx_v)
    pltpu.sync_copy(iids_hbm.at[pl.ds(base, bpw)], iidx_v)
    for j in range(bpw // 16):
        s = pl.ds(16 * j, 16)
        u = uidx_v[s]
        i = iidx_v[s]
        uhalf_v[s] = ((u >> 11) << 10) | (u & 1023)
        ihalf_v[s] = ((i >> 11) << 10) | (i & 1023)
        upar_v[s] = ((u >> 10) & 1) << 6
        ipar_v[s] = ((i >> 10) & 1) << 6

    for c in range(bpw // _CH):
        off = c * _CH
        cu = pltpu.async_copy(t1_hbm.at[uhalf_v.at[pl.ds(off, _CH)]],
                              urows_v, sem_u)
        ci = pltpu.async_copy(t2_hbm.at[ihalf_v.at[pl.ds(off, _CH)]],
                              irows_v, sem_i)
        cu.wait()
        ci.wait()

        def repack(j, carry):
            rows = lax.broadcasted_iota(jnp.int32, (16,), 0) + 16 * j
            pu = upar_v[pl.ds(off + 16 * j, 16)]
            pi = ipar_v[pl.ds(off + 16 * j, 16)]
            for col in range(EMB):
                cc = jnp.full((16,), col, jnp.int32)
                vu = plsc.load_gather(urows_v, [rows, pu + col])
                plsc.store_scatter(cat_v, [rows, cc], vu)
                vi = plsc.load_gather(irows_v, [rows, pi + col])
                plsc.store_scatter(cat_v, [rows, cc + EMB], vi)
            return carry

        lax.fori_loop(0, _CH // 16, repack, 0)
        pltpu.sync_copy(cat_v, x_hbm.at[pl.ds(base + off, _CH)])


def _sc_gather(user_ids, item_ids, t1, t2):
    bpw = BATCH // _NW
    mesh = plsc.VectorSubcoreMesh(core_axis_name="c", subcore_axis_name="s")
    out_type = jax.ShapeDtypeStruct((BATCH, 2 * EMB), jnp.float32)
    scratch = [
        pltpu.VMEM((bpw,), jnp.int32),
        pltpu.VMEM((bpw,), jnp.int32),
        pltpu.VMEM((bpw,), jnp.int32),
        pltpu.VMEM((bpw,), jnp.int32),
        pltpu.VMEM((bpw,), jnp.int32),
        pltpu.VMEM((bpw,), jnp.int32),
        pltpu.VMEM((_CH, 2 * EMB), jnp.float32),
        pltpu.VMEM((_CH, 2 * EMB), jnp.float32),
        pltpu.VMEM((_CH, 2 * EMB), jnp.float32),
        pltpu.SemaphoreType.DMA,
        pltpu.SemaphoreType.DMA,
    ]
    k = pl.kernel(
        functools.partial(_gather_body, bpw=bpw),
        out_type=out_type,
        mesh=mesh,
        scratch_types=scratch,
        compiler_params=pltpu.CompilerParams(
            use_tc_tiling_on_sc=False,
            needs_layout_passes=False,
        ),
    )
    return k(user_ids, item_ids, t1, t2)


def _layernorm(x, g, b, eps=1e-5):
    mu = jnp.mean(x, axis=-1, keepdims=True)
    var = jnp.mean((x - mu) ** 2, axis=-1, keepdims=True)
    return (x - mu) / jnp.sqrt(var + eps) * g + b


def _gelu(x):
    return 0.5 * x * (1.0 + lax.erf(x * (2.0 ** -0.5)))


def _mlp_body(xin, W0, b0, g0, beta0, W1, b1, g1, beta1,
              W2, b2, g2, beta2, W_out, b_out, out):
    dot = functools.partial(jnp.dot, preferred_element_type=jnp.float32)
    x = dot(xin[...], W0[...]) + b0[...]
    x = _gelu(_layernorm(x, g0[...], beta0[...]))
    x = dot(x, W1[...]) + b1[...]
    x = _gelu(_layernorm(x, g1[...], beta1[...]))
    x = dot(x, W2[...]) + b2[...]
    x = _gelu(_layernorm(x, g2[...], beta2[...]))
    out[...] = dot(x, W_out[...]) + b_out[...]


def _tc_mlp(x, W0, b0, g0, beta0, W1, b1, g1, beta1,
            W2, b2, g2, beta2, W_out, b_out):
    blk = 2048
    grid = (BATCH // blk,)

    def full_spec(a):
        return pl.BlockSpec(a.shape, lambda i: (0,) * a.ndim)

    b0r, g0r, beta0r = (a.reshape(1, -1) for a in (b0, g0, beta0))
    b1r, g1r, beta1r = (a.reshape(1, -1) for a in (b1, g1, beta1))
    b2r, g2r, beta2r = (a.reshape(1, -1) for a in (b2, g2, beta2))
    b_outr = b_out.reshape(1, -1)

    args = (x, W0, b0r, g0r, beta0r, W1, b1r, g1r, beta1r,
            W2, b2r, g2r, beta2r, W_out, b_outr)
    in_specs = ([pl.BlockSpec((blk, 2 * EMB), lambda i: (i, 0))]
                + [full_spec(a) for a in args[1:]])
    return pl.pallas_call(
        _mlp_body,
        grid=grid,
        in_specs=in_specs,
        out_specs=pl.BlockSpec((blk, 1), lambda i: (i, 0)),
        out_shape=jax.ShapeDtypeStruct((BATCH, 1), jnp.float32),
    )(*args)


def kernel(user_ids, item_ids, user_table, item_table,
           W0, b0, g0, beta0, W1, b1, g1, beta1, W2, b2, g2, beta2,
           W_out, b_out):
    t1, t2 = _detranspose(user_table.T, item_table.T)
    x = _sc_gather(user_ids.astype(jnp.int32), item_ids.astype(jnp.int32),
                   t1, t2)
    return _tc_mlp(x, W0, b0, g0, beta0, W1, b1, g1, beta1,
                   W2, b2, g2, beta2, W_out, b_out)
